# baseline (speedup 1.0000x reference)
import functools

import jax
import jax.numpy as jnp
from jax import lax
from jax.experimental import pallas as pl
from jax.experimental.pallas import tpu as pltpu
from jax.experimental.pallas import tpu_sc as plsc

M = 8
K = 8192
D = 256

NC = 2
NS = 16
NW = NC * NS

CB = 16
IDX = CB * M
LANES = 16


def _body(b_per_w, idx_hbm, table_hbm, out_hbm, idx_v, rows_v, out_v, sem):
    wid = lax.axis_index("s") * NC + lax.axis_index("c")
    n_chunks = b_per_w // CB
    pat = (lax.iota(jnp.int32, LANES) % M) * K

    def chunk_body(c, carry):
        base = wid * b_per_w + c * CB
        pltpu.sync_copy(idx_hbm.at[pl.ds(base * M, IDX)], idx_v)
        for g in range(IDX // LANES):
            sl = pl.ds(g * LANES, LANES)
            idx_v[sl] = idx_v[sl] + pat
        pltpu.async_copy(table_hbm.at[idx_v], rows_v, sem).wait()

        def row_body(r, carry2):
            for d in range(D // LANES):
                sl = pl.ds(d * LANES, LANES)
                acc = rows_v[r * M, sl]
                for m in range(1, M):
                    acc = acc + rows_v[r * M + m, sl]
                out_v[r, sl] = acc
            return carry2

        lax.fori_loop(0, CB, row_body, 0, unroll=False)
        pltpu.sync_copy(out_v, out_hbm.at[pl.ds(base, CB)])
        return carry

    lax.fori_loop(0, n_chunks, chunk_body, 0, unroll=False)


@functools.partial(jax.jit, static_argnames=())
def kernel(pickedIndices, codebook):
    B = pickedIndices.shape[0]
    b_per_w = B // NW
    table = codebook.reshape(M * K, D)
    idx_flat = pickedIndices.reshape(B * M)

    mesh = plsc.VectorSubcoreMesh(core_axis_name="c", subcore_axis_name="s")
    run = pl.kernel(
        functools.partial(_body, b_per_w),
        out_type=jax.ShapeDtypeStruct((B, D), jnp.float32),
        mesh=mesh,
        scratch_types=[
            pltpu.VMEM((IDX,), jnp.int32),
            pltpu.VMEM((IDX, D), jnp.float32),
            pltpu.VMEM((CB, D), jnp.float32),
            pltpu.SemaphoreType.DMA,
        ],
    )
    return run(idx_flat, table)


# speedup vs baseline: 1.5732x; 1.5732x over previous
import functools

import jax
import jax.numpy as jnp
from jax import lax
from jax.experimental import pallas as pl
from jax.experimental.pallas import tpu as pltpu
from jax.experimental.pallas import tpu_sc as plsc

M = 8
K = 8192
D = 256

NC = 2
NS = 16
NW = NC * NS

CB = 16
IDX = CB * M
LANES = 16


def _body(b_per_w, idx_hbm, table_hbm, out_hbm,
          idx_v, rows0, rows1, out0, out1, sg0, sg1, so0, so1):
    wid = lax.axis_index("s") * NC + lax.axis_index("c")
    base = wid * b_per_w
    n_chunks = b_per_w // CB
    n_pairs = n_chunks // 2

    pltpu.sync_copy(idx_hbm.at[pl.ds(base * M, b_per_w * M)], idx_v)

    rows = (rows0, rows1)
    outs = (out0, out1)
    sgs = (sg0, sg1)
    sos = (so0, so1)

    def fire_gather(c, p):
        pltpu.async_copy(
            table_hbm.at[idx_v.at[pl.ds(c * IDX, IDX)]], rows[p], sgs[p])

    def wait_gather(p):
        pltpu.make_async_copy(
            table_hbm.at[idx_v.at[pl.ds(0, IDX)]], rows[p], sgs[p]).wait()

    def out_slice(c):
        return out_hbm.at[pl.ds(base + c * CB, CB)]

    def wait_out(p):
        pltpu.make_async_copy(outs[p], out_slice(0), sos[p]).wait()

    def compute(p):
        rv, ov = rows[p], outs[p]

        def row_body(r, carry):
            for d in range(D // LANES):
                sl = pl.ds(d * LANES, LANES)
                acc = rv[r * M, sl]
                for m in range(1, M):
                    acc = acc + rv[r * M + m, sl]
                ov[r, sl] = acc
            return carry

        lax.fori_loop(0, CB, row_body, 0, unroll=False)

    fire_gather(0, 0)

    def pair_body(i, carry):
        a = 2 * i
        fire_gather(a + 1, 1)
        wait_gather(0)

        @pl.when(i > 0)
        def _():
            wait_out(0)

        compute(0)
        pltpu.async_copy(out0, out_slice(a), so0)

        @pl.when(i < n_pairs - 1)
        def _():
            fire_gather(a + 2, 0)

        wait_gather(1)

        @pl.when(i > 0)
        def _():
            wait_out(1)

        compute(1)
        pltpu.async_copy(out1, out_slice(a + 1), so1)
        return carry

    lax.fori_loop(0, n_pairs, pair_body, 0, unroll=False)
    wait_out(0)
    wait_out(1)


@jax.jit
def kernel(pickedIndices, codebook):
    B = pickedIndices.shape[0]
    b_per_w = B // NW
    table = codebook.reshape(M * K, D)
    idx_flat = (pickedIndices + jnp.arange(M, dtype=jnp.int32)[None, :] * K
                ).reshape(B * M)

    mesh = plsc.VectorSubcoreMesh(core_axis_name="c", subcore_axis_name="s")
    run = pl.kernel(
        functools.partial(_body, b_per_w),
        out_type=jax.ShapeDtypeStruct((B, D), jnp.float32),
        mesh=mesh,
        scratch_types=[
            pltpu.VMEM((b_per_w * M,), jnp.int32),
            pltpu.VMEM((IDX, D), jnp.float32),
            pltpu.VMEM((IDX, D), jnp.float32),
            pltpu.VMEM((CB, D), jnp.float32),
            pltpu.VMEM((CB, D), jnp.float32),
            pltpu.SemaphoreType.DMA,
            pltpu.SemaphoreType.DMA,
            pltpu.SemaphoreType.DMA,
            pltpu.SemaphoreType.DMA,
        ],
    )
    return run(idx_flat, table)


# speedup vs baseline: 2.2605x; 1.4369x over previous
import functools

import jax
import jax.numpy as jnp
from jax import lax
from jax.experimental import pallas as pl
from jax.experimental.pallas import tpu as pltpu
from jax.experimental.pallas import tpu_sc as plsc

M = 8
K = 8192
D = 256

NC = 2
NS = 16
NW = NC * NS

CB = 16
IDX = CB * M
LANES = 16


def _body(b_per_w, idx_hbm, table_hbm, out_hbm,
          idx_v, rows0, rows1, out0, out1, sg0, sg1, so0, so1):
    wid = lax.axis_index("s") * NC + lax.axis_index("c")
    base = wid * b_per_w
    n_chunks = b_per_w // CB
    n_pairs = n_chunks // 2

    pltpu.sync_copy(idx_hbm.at[pl.ds(base * M, b_per_w * M)], idx_v)

    rows = (rows0, rows1)
    outs = (out0, out1)
    sgs = (sg0, sg1)
    sos = (so0, so1)

    def fire_gather(c, p):
        pltpu.async_copy(
            table_hbm.at[idx_v.at[pl.ds(c * IDX, IDX)]], rows[p], sgs[p])

    def wait_gather(p):
        pltpu.make_async_copy(
            table_hbm.at[idx_v.at[pl.ds(0, IDX)]], rows[p], sgs[p]).wait()

    def out_slice(c):
        return out_hbm.at[pl.ds(base + c * CB, CB)]

    def wait_out(p):
        pltpu.make_async_copy(outs[p], out_slice(0), sos[p]).wait()

    def compute(p):
        rv, ov = rows[p], outs[p]

        @plsc.parallel_loop(0, CB, step=1, unroll=2)
        def row_body(r):
            for d in range(D // LANES):
                sl = pl.ds(d * LANES, LANES)
                acc = rv[r * M, sl]
                for m in range(1, M):
                    acc = acc + rv[r * M + m, sl]
                ov[r, sl] = acc

    fire_gather(0, 0)

    def pair_body(i, carry):
        a = 2 * i
        fire_gather(a + 1, 1)
        wait_gather(0)

        @pl.when(i > 0)
        def _():
            wait_out(0)

        compute(0)
        pltpu.async_copy(out0, out_slice(a), so0)

        @pl.when(i < n_pairs - 1)
        def _():
            fire_gather(a + 2, 0)

        wait_gather(1)

        @pl.when(i > 0)
        def _():
            wait_out(1)

        compute(1)
        pltpu.async_copy(out1, out_slice(a + 1), so1)
        return carry

    lax.fori_loop(0, n_pairs, pair_body, 0, unroll=False)
    wait_out(0)
    wait_out(1)


@jax.jit
def kernel(pickedIndices, codebook):
    B = pickedIndices.shape[0]
    b_per_w = B // NW
    table = codebook.reshape(M * K, D)
    idx_flat = (pickedIndices + jnp.arange(M, dtype=jnp.int32)[None, :] * K
                ).reshape(B * M)

    mesh = plsc.VectorSubcoreMesh(core_axis_name="c", subcore_axis_name="s")
    run = pl.kernel(
        functools.partial(_body, b_per_w),
        out_type=jax.ShapeDtypeStruct((B, D), jnp.float32),
        mesh=mesh,
        scratch_types=[
            pltpu.VMEM((b_per_w * M,), jnp.int32),
            pltpu.VMEM((IDX, D), jnp.float32),
            pltpu.VMEM((IDX, D), jnp.float32),
            pltpu.VMEM((CB, D), jnp.float32),
            pltpu.VMEM((CB, D), jnp.float32),
            pltpu.SemaphoreType.DMA,
            pltpu.SemaphoreType.DMA,
            pltpu.SemaphoreType.DMA,
            pltpu.SemaphoreType.DMA,
        ],
    )
    return run(idx_flat, table)
